# baseline (speedup 1.0000x reference)
import functools

import jax
import jax.numpy as jnp
from jax import lax
from jax.experimental import pallas as pl
from jax.experimental.pallas import tpu as pltpu
from jax.experimental.pallas import tpu_sc as plsc

B, K, L, NUM_CLASSES, D = 16, 8, 2048, 1024, 128
NC, NS = 2, 16
KH = K // NC
THALF = KH * NUM_CLASSES
G = B * K
CH = 128
GW = G // NC
ROWS = B * K * L


def _emb_body(tables_hbm, idx_hbm, frame_hbm, out_hbm,
              tables_sp, frame_sp, idx_v, rows0, rows1, rows2, rows3,
              fsem0, fsem1, fsem2, fsem3,
              gsem0, gsem1, gsem2, gsem3, ssem0, ssem1, ssem2, ssem3):
    rows = (rows0, rows1, rows2, rows3)
    fsems = (fsem0, fsem1, fsem2, fsem3)
    gsems = (gsem0, gsem1, gsem2, gsem3)
    ssems = (ssem0, ssem1, ssem2, ssem3)

    c = lax.axis_index("c")
    p = lax.axis_index("s")

    tsl = THALF // NS
    pltpu.sync_copy(tables_hbm.at[pl.ds(c * THALF + p * tsl, tsl)],
                    tables_sp.at[pl.ds(p * tsl, tsl)])
    pltpu.sync_copy(frame_hbm.at[pl.ds(p * CH, CH)],
                    frame_sp.at[pl.ds(p * CH, CH)])

    pltpu.sync_copy(idx_hbm.at[:, pl.ds(p * CH, CH)], idx_v)

    def grow(j):
        return 8 * (j // 4) + 4 * c + lax.rem(j, 4)

    def adj(j, carry):
        r = grow(j)
        off = jnp.full((16,), lax.rem(j, 4) * NUM_CLASSES, jnp.int32)
        for v in range(CH // 16):
            sl = (r, pl.ds(v * 16, 16))
            idx_v[sl] = idx_v[sl] + off
        return carry

    lax.fori_loop(0, GW, adj, 0)

    plsc.subcore_barrier()

    fchunk = frame_sp.at[pl.ds(p * CH, CH)]

    def fill_issue(b):
        pltpu.async_copy(fchunk, rows[b], fsems[b])

    def fill_wait(b):
        pltpu.make_async_copy(fchunk, rows[b], fsems[b]).wait()

    def gadd_issue(j, b):
        pltpu.async_copy(tables_sp.at[idx_v.at[grow(j)]], rows[b], gsems[b],
                         add=True)

    def gadd_wait(j, b):
        pltpu.make_async_copy(
            tables_sp.at[idx_v.at[grow(j)]], rows[b], gsems[b]).wait()

    def store_issue(j, b):
        base = grow(j) * L + p * CH
        pltpu.async_copy(rows[b], out_hbm.at[pl.ds(base, CH)], ssems[b])

    def store_wait(b):
        pltpu.make_async_copy(rows[b], out_hbm.at[pl.ds(0, CH)],
                              ssems[b]).wait()

    fill_issue(0)
    fill_issue(1)
    fill_wait(0)
    gadd_issue(0, 0)

    def step(i, carry):
        for u in range(4):
            j = 4 * i + u
            b = u
            nb = (u + 2) % 4
            fb = (u + 1) % 4
            gadd_wait(j, b)
            store_issue(j, b)

            if u < 2:
                @pl.when(i >= 1)
                def _():
                    store_wait(nb)
                fill_issue(nb)
            else:
                @pl.when(i < GW // 4 - 1)
                def _():
                    store_wait(nb)
                    fill_issue(nb)

            if u < 3:
                fill_wait(fb)
                gadd_issue(j + 1, fb)
            else:
                @pl.when(i < GW // 4 - 1)
                def _():
                    fill_wait(fb)
                    gadd_issue(j + 1, fb)

        return carry

    lax.fori_loop(0, GW // 4, step, 0)
    for b in range(4):
        store_wait(b)


@functools.partial(
    pl.kernel,
    mesh=plsc.VectorSubcoreMesh(core_axis_name="c", subcore_axis_name="s"),
    out_type=jax.ShapeDtypeStruct((ROWS, D), jnp.float32),
    scratch_types=[
        pltpu.VMEM_SHARED((THALF, D), jnp.float32),
        pltpu.VMEM_SHARED((L, D), jnp.float32),
        pltpu.VMEM((G, CH), jnp.int32),
        pltpu.VMEM((CH, D), jnp.float32),
        pltpu.VMEM((CH, D), jnp.float32),
        pltpu.VMEM((CH, D), jnp.float32),
        pltpu.VMEM((CH, D), jnp.float32),
        pltpu.SemaphoreType.DMA,
        pltpu.SemaphoreType.DMA,
        pltpu.SemaphoreType.DMA,
        pltpu.SemaphoreType.DMA,
        pltpu.SemaphoreType.DMA,
        pltpu.SemaphoreType.DMA,
        pltpu.SemaphoreType.DMA,
        pltpu.SemaphoreType.DMA,
        pltpu.SemaphoreType.DMA,
        pltpu.SemaphoreType.DMA,
        pltpu.SemaphoreType.DMA,
        pltpu.SemaphoreType.DMA,
    ],
)
def _emb_kernel(tables_hbm, idx_hbm, frame_hbm, out_hbm,
                tables_sp, frame_sp, idx_v, rows0, rows1, rows2, rows3,
                fsem0, fsem1, fsem2, fsem3,
                gsem0, gsem1, gsem2, gsem3, ssem0, ssem1, ssem2, ssem3):
    _emb_body(tables_hbm, idx_hbm, frame_hbm, out_hbm,
              tables_sp, frame_sp, idx_v, rows0, rows1, rows2, rows3,
              fsem0, fsem1, fsem2, fsem3,
              gsem0, gsem1, gsem2, gsem3, ssem0, ssem1, ssem2, ssem3)


@jax.jit
def kernel(index, content_tables, frame_table):
    tables = content_tables.reshape(K * NUM_CLASSES, D)
    idx = index.reshape(G, L).astype(jnp.int32)
    out = _emb_kernel(tables, idx, frame_table[:L])
    return out.reshape(B, K, L, D)


# speedup vs baseline: 1.2127x; 1.2127x over previous
import functools

import jax
import jax.numpy as jnp
from jax import lax
from jax.experimental import pallas as pl
from jax.experimental.pallas import tpu as pltpu
from jax.experimental.pallas import tpu_sc as plsc

B, K, L, NUM_CLASSES, D = 16, 8, 2048, 1024, 128
NC, NS = 2, 16
KH = K // NC
THALF = KH * NUM_CLASSES
G = B * K
CH = 128
HC = CH // 2
GW = G // NC
NSTEP = 2 * GW
NB = 8
LOOK = 4
ROWS = B * K * L


def _emb_body(tables_hbm, idx_hbm, frame_hbm, out_hbm,
              tables_sp, idx_v, frame_v, rowbufs, gsems, ssems):
    c = lax.axis_index("c")
    p = lax.axis_index("s")

    tsl = THALF // NS
    pltpu.sync_copy(tables_hbm.at[pl.ds(c * THALF + p * tsl, tsl)],
                    tables_sp.at[pl.ds(p * tsl, tsl)])

    pltpu.sync_copy(idx_hbm.at[:, pl.ds(p * CH, CH)], idx_v)
    pltpu.sync_copy(frame_hbm.at[pl.ds(p * CH, CH)], frame_v)

    def grow(jj):
        return 8 * (jj // 4) + 4 * c + lax.rem(jj, 4)

    def adj(jj, carry):
        r = grow(jj)
        off = jnp.full((16,), lax.rem(jj, 4) * NUM_CLASSES, jnp.int32)
        for v in range(CH // 16):
            sl = (r, pl.ds(v * 16, 16))
            idx_v[sl] = idx_v[sl] + off
        return carry

    lax.fori_loop(0, GW, adj, 0)

    plsc.subcore_barrier()

    def gather_issue(j, b):
        jj, h = j // 2, lax.rem(j, 2)
        idx_sl = idx_v.at[grow(jj), pl.ds(h * HC, HC)]
        pltpu.async_copy(tables_sp.at[idx_sl], rowbufs[b], gsems[b])

    def gather_wait(j, b):
        jj, h = j // 2, lax.rem(j, 2)
        idx_sl = idx_v.at[grow(jj), pl.ds(h * HC, HC)]
        pltpu.make_async_copy(tables_sp.at[idx_sl], rowbufs[b],
                              gsems[b]).wait()

    def store_issue(j, b):
        jj, h = j // 2, lax.rem(j, 2)
        base = grow(jj) * L + p * CH + h * HC
        pltpu.async_copy(rowbufs[b], out_hbm.at[pl.ds(base, HC)], ssems[b])

    def store_wait(b):
        pltpu.make_async_copy(rowbufs[b], out_hbm.at[pl.ds(0, HC)],
                              ssems[b]).wait()

    for b in range(LOOK):
        gather_issue(b, b)

    def step(i, carry):
        for u in range(NB):
            j = NB * i + u
            b = u
            nb = (u + LOOK) % NB
            gather_wait(j, b)

            if u < LOOK:
                @pl.when(i >= 1)
                def _():
                    store_wait(nb)
                gather_issue(j + LOOK, nb)
            else:
                @pl.when(i < NSTEP // NB - 1)
                def _():
                    store_wait(nb)
                    gather_issue(j + LOOK, nb)

            @plsc.parallel_loop(0, HC, step=1, unroll=4)
            def add_rows(r):
                jh = lax.rem(j, 2) * HC
                for v in range(D // 16):
                    plsc.addupdate(rowbufs[b].at[(r, pl.ds(v * 16, 16))],
                                   frame_v[jh + r, pl.ds(v * 16, 16)])

            store_issue(j, b)
        return carry

    lax.fori_loop(0, NSTEP // NB, step, 0)
    for b in range(NB):
        store_wait(b)


@functools.partial(
    pl.kernel,
    mesh=plsc.VectorSubcoreMesh(core_axis_name="c", subcore_axis_name="s"),
    out_type=jax.ShapeDtypeStruct((ROWS, D), jnp.float32),
    scratch_types=(
        [pltpu.VMEM_SHARED((THALF, D), jnp.float32),
         pltpu.VMEM((G, CH), jnp.int32),
         pltpu.VMEM((CH, D), jnp.float32)]
        + [pltpu.VMEM((HC, D), jnp.float32)] * NB
        + [pltpu.SemaphoreType.DMA] * (2 * NB)
    ),
)
def _emb_kernel(tables_hbm, idx_hbm, frame_hbm, out_hbm,
                tables_sp, idx_v, frame_v, *bufs_and_sems):
    rowbufs = bufs_and_sems[:NB]
    gsems = bufs_and_sems[NB:2 * NB]
    ssems = bufs_and_sems[2 * NB:3 * NB]
    _emb_body(tables_hbm, idx_hbm, frame_hbm, out_hbm,
              tables_sp, idx_v, frame_v, rowbufs, gsems, ssems)


@jax.jit
def kernel(index, content_tables, frame_table):
    tables = content_tables.reshape(K * NUM_CLASSES, D)
    idx = index.reshape(G, L).astype(jnp.int32)
    out = _emb_kernel(tables, idx, frame_table[:L])
    return out.reshape(B, K, L, D)
